# baseline (speedup 1.0000x reference)
import functools

import jax
import jax.numpy as jnp
from jax import lax
from jax.experimental import pallas as pl
from jax.experimental.pallas import tpu as pltpu
from jax.experimental.pallas import tpu_sc as plsc

N = 10000
E = 320000
D = 128
HID = 128
NL = 4
EPS = 1e-8

NT = 10240
EP = 327680
GRP = EP // 128
NC = 2
NS = 16
NW = NC * NS
GPW = GRP // NW
GPC = GRP // NC
RPT = NT // NS
BN = 512
BE = 1024
GPB = BE // 128
WID2 = HID + 16

_F32 = jnp.float32


def _silu(x):
    return x * jax.nn.sigmoid(x)


def _dot(a, b):
    return jnp.dot(a, b, preferred_element_type=_F32)


def _sc_gather(a, b, cflat, rowg, colg):
    mesh = plsc.VectorSubcoreMesh(core_axis_name="c", subcore_axis_name="s")

    @functools.partial(
        pl.kernel,
        out_type=[jax.ShapeDtypeStruct((EP, HID), _F32),
                  jax.ShapeDtypeStruct((EP, HID), _F32),
                  jax.ShapeDtypeStruct((GRP, 128), _F32),
                  jax.ShapeDtypeStruct((GRP, 128), _F32),
                  jax.ShapeDtypeStruct((GRP, 128), _F32)],
        mesh=mesh,
        scratch_types=[pltpu.VMEM((GPW, 128), jnp.int32),
                       pltpu.VMEM((GPW, 128), jnp.int32),
                       [pltpu.VMEM((128, HID), _F32) for _ in range(2)],
                       [pltpu.VMEM((128, HID), _F32) for _ in range(2)],
                       [[pltpu.VMEM((128,), jnp.int32) for _ in range(6)]
                        for _ in range(2)],
                       [[pltpu.VMEM((128,), _F32) for _ in range(6)]
                        for _ in range(2)],
                       [pltpu.VMEM((GPW, 128), _F32) for _ in range(3)],
                       [pltpu.SemaphoreType.DMA for _ in range(2)],
                       [pltpu.SemaphoreType.DMA for _ in range(2)],
                       [pltpu.SemaphoreType.DMA for _ in range(2)],
                       [pltpu.SemaphoreType.DMA for _ in range(2)],
                       [pltpu.SemaphoreType.DMA for _ in range(2)]],
    )
    def k(ar, br, cfr, rowr, colr, xa, xb, dxh, dyh, dzh,
          idxr, idxc, bufa, bufb, ixs, cbs, dacc,
          sema, semb, semc, semsa, semsb):
        wid = lax.axis_index("s") * NC + lax.axis_index("c")
        g0 = wid * GPW
        pltpu.sync_copy(rowr.at[pl.ds(g0, GPW)], idxr)
        pltpu.sync_copy(colr.at[pl.ds(g0, GPW)], idxc)
        dxa, dya, dza = dacc

        def fire(g, p):
            ixr0, ixr1, ixr2, ixc0, ixc1, ixc2 = ixs[p]
            for j in range(8):
                sl = pl.ds(j * 16, 16)
                r4 = idxr[g, sl] * 4
                c4 = idxc[g, sl] * 4
                ixr0[sl] = r4
                ixr1[sl] = r4 + 1
                ixr2[sl] = r4 + 2
                ixc0[sl] = c4
                ixc1[sl] = c4 + 1
                ixc2[sl] = c4 + 2
            pltpu.async_copy(ar.at[idxr.at[g]], bufa[p], sema[p])
            pltpu.async_copy(br.at[idxc.at[g]], bufb[p], semb[p])
            brx, bry, brz, bcx, bcy, bcz = cbs[p]
            pltpu.async_copy(cfr.at[ixr0], brx, semc[p])
            pltpu.async_copy(cfr.at[ixr1], bry, semc[p])
            pltpu.async_copy(cfr.at[ixr2], brz, semc[p])
            pltpu.async_copy(cfr.at[ixc0], bcx, semc[p])
            pltpu.async_copy(cfr.at[ixc1], bcy, semc[p])
            pltpu.async_copy(cfr.at[ixc2], bcz, semc[p])

        fire(0, 0)

        def body(s, carry):
            for p in range(2):
                g = s * 2 + p
                q = 1 - p

                @pl.when(g >= 1)
                def _():
                    pltpu.make_async_copy(bufa[q], xa.at[pl.ds(0, 128)],
                                          semsa[q]).wait()
                    pltpu.make_async_copy(bufb[q], xb.at[pl.ds(0, 128)],
                                          semsb[q]).wait()

                @pl.when(g + 1 < GPW)
                def _():
                    fire(g + 1, q)

                pltpu.make_async_copy(ar.at[idxr.at[g]], bufa[p],
                                      sema[p]).wait()
                pltpu.make_async_copy(br.at[idxc.at[g]], bufb[p],
                                      semb[p]).wait()
                brx, bry, brz, bcx, bcy, bcz = cbs[p]
                for cb2 in cbs[p]:
                    pltpu.make_async_copy(cfr.at[pl.ds(0, 128)], cb2,
                                          semc[p]).wait()
                for j in range(8):
                    sl = pl.ds(j * 16, 16)
                    dxa[g, sl] = brx[sl] - bcx[sl]
                    dya[g, sl] = bry[sl] - bcy[sl]
                    dza[g, sl] = brz[sl] - bcz[sl]
                base = (g0 + g) * 128
                pltpu.async_copy(bufa[p], xa.at[pl.ds(base, 128)], semsa[p])
                pltpu.async_copy(bufb[p], xb.at[pl.ds(base, 128)], semsb[p])
            return carry

        lax.fori_loop(0, GPW // 2, body, 0)
        pltpu.make_async_copy(bufa[1], xa.at[pl.ds(0, 128)], semsa[1]).wait()
        pltpu.make_async_copy(bufb[1], xb.at[pl.ds(0, 128)], semsb[1]).wait()
        pltpu.sync_copy(dxa, dxh.at[pl.ds(g0, GPW)])
        pltpu.sync_copy(dya, dyh.at[pl.ds(g0, GPW)])
        pltpu.sync_copy(dza, dzh.at[pl.ds(g0, GPW)])

    return k(a, b, cflat, rowg, colg)


NT4 = NT * 4
ZPT = NT4 // NS
GRP8 = GRP // 8
SPW = GPW // 8


def _sc_scatter(yf, txp, typ, tzp, rowg, zrows, zflat):
    mesh = plsc.VectorSubcoreMesh(core_axis_name="c", subcore_axis_name="s")

    @functools.partial(
        pl.kernel,
        out_type=[jax.ShapeDtypeStruct((NC, NT, HID), _F32),
                  jax.ShapeDtypeStruct((NC, NT4), _F32)],
        mesh=mesh,
        scratch_types=[pltpu.VMEM((8, 128), jnp.int32),
                       [pltpu.VMEM((128, HID), _F32) for _ in range(2)],
                       [pltpu.VMEM((8, 128), _F32) for _ in range(3)],
                       [pltpu.VMEM((128,), jnp.int32) for _ in range(3)],
                       pltpu.VMEM_SHARED((NT, HID), _F32),
                       pltpu.VMEM_SHARED((NT4,), _F32),
                       [pltpu.SemaphoreType.DMA for _ in range(2)],
                       [pltpu.SemaphoreType.DMA for _ in range(2)]],
    )
    def k(yr, txr, tyr, tzr, rowr, zr, zfr, outf, outc,
          idx8, buf, tvs, ixs, accf, accc, semr, semw):
        cid = lax.axis_index("c")
        sid = lax.axis_index("s")
        txv, tyv, tzv = tvs
        ix0, ix1, ix2 = ixs
        for j in range(RPT // 128):
            pltpu.sync_copy(zr, accf.at[pl.ds(sid * RPT + j * 128, 128)])
        pltpu.sync_copy(zfr, accc.at[pl.ds(sid * ZPT, ZPT)])
        plsc.subcore_barrier()

        g0 = cid * GPC + sid * GPW
        s0 = g0 // 8
        pltpu.async_copy(yr.at[pl.ds(g0 * 128, 128)], buf[0], semr[0])

        def body(s, carry):
            @pl.when(s >= 1)
            def _():
                pltpu.make_async_copy(buf[1], accf.at[pl.ds(0, 128)],
                                      semw[1]).wait()

            pltpu.sync_copy(rowr.at[pl.ds(g0 + s * 8, 8)], idx8)
            pltpu.sync_copy(txr.at[s0 + s], txv)
            pltpu.sync_copy(tyr.at[s0 + s], tyv)
            pltpu.sync_copy(tzr.at[s0 + s], tzv)
            for j8 in range(8):
                g = s * 8 + j8
                p = j8 % 2
                q = 1 - p

                if j8 >= 1:
                    pltpu.make_async_copy(buf[q], accf.at[pl.ds(0, 128)],
                                          semw[q]).wait()

                @pl.when(g + 1 < GPW)
                def _():
                    pltpu.async_copy(yr.at[pl.ds((g0 + g + 1) * 128, 128)],
                                     buf[q], semr[q])

                pltpu.make_async_copy(yr.at[pl.ds(0, 128)], buf[p],
                                      semr[p]).wait()
                pltpu.async_copy(buf[p], accf.at[idx8.at[j8]], semw[p],
                                add=True)
                for j in range(8):
                    sl = pl.ds(j * 16, 16)
                    r4 = idx8[j8, sl] * 4
                    ix0[sl] = r4
                    ix1[sl] = r4 + 1
                    ix2[sl] = r4 + 2
                pltpu.sync_copy(txv.at[j8], accc.at[ix0], add=True)
                pltpu.sync_copy(tyv.at[j8], accc.at[ix1], add=True)
                pltpu.sync_copy(tzv.at[j8], accc.at[ix2], add=True)
            return carry

        lax.fori_loop(0, SPW, body, 0)
        pltpu.make_async_copy(buf[1], accf.at[pl.ds(0, 128)], semw[1]).wait()
        plsc.subcore_barrier()
        for j in range(RPT // 128):
            r0 = sid * RPT + j * 128
            pltpu.sync_copy(accf.at[pl.ds(r0, 128)],
                            outf.at[cid, pl.ds(r0, 128)])
        pltpu.sync_copy(accc.at[pl.ds(sid * ZPT, ZPT)],
                        outc.at[cid, pl.ds(sid * ZPT, ZPT)])

    return k(yf, txp, typ, tzp, rowg, zrows, zflat)


def _full(shape):
    return pl.BlockSpec(shape, lambda i: (0,) * len(shape))


def _col_from_planar(p, ei_mask):
    rep = jnp.concatenate(
        [jnp.broadcast_to(p[s:s + 1, :], (128, 128)) for s in range(GPB)],
        axis=0)
    return jnp.sum(jnp.where(ei_mask, rep, 0.0), axis=1, keepdims=True)


def _tc_pre(h0p, einT, einb, w1aT, w1bT):
    def body(h_ref, wt, bt, wa, wb, h1_ref, a_ref, b_ref):
        h1 = _dot(h_ref[...], wt[...]) + bt[...]
        h1_ref[...] = h1
        a_ref[...] = _dot(h1, wa[...])
        b_ref[...] = _dot(h1, wb[...])

    return pl.pallas_call(
        body,
        grid=(NT // BN,),
        in_specs=[pl.BlockSpec((BN, D), lambda i: (i, 0)),
                  _full((D, HID)), _full((1, HID)),
                  _full((HID, HID)), _full((HID, HID))],
        out_specs=[pl.BlockSpec((BN, HID), lambda i: (i, 0)),
                   pl.BlockSpec((BN, HID), lambda i: (i, 0)),
                   pl.BlockSpec((BN, HID), lambda i: (i, 0))],
        out_shape=[jax.ShapeDtypeStruct((NT, HID), _F32),
                   jax.ShapeDtypeStruct((NT, HID), _F32),
                   jax.ShapeDtypeStruct((NT, HID), _F32)],
    )(h0p, einT, einb, w1aT, w1bT)


def _tc_edge(xa, xb, dxp, dyp, dzp, w1c, b1, w2T, b2, cw1T, cb1, cw2):
    def body(xar, xbr, dxr, dyr, dzr, w1cr, b1r, w2r, b2r, cw1r, cb1r,
             cw2r, yr, txr, tyr, tzr):
        li = lax.broadcasted_iota(jnp.int32, (BE, 128), 1)
        ei = lax.broadcasted_iota(jnp.int32, (BE, 128), 0)
        msk = li == (ei % 128)
        dx = _col_from_planar(dxr[...], msk)
        dy = _col_from_planar(dyr[...], msk)
        dz = _col_from_planar(dzr[...], msk)
        rad = dx * dx + dy * dy + dz * dz
        norm = jnp.sqrt(rad) + EPS
        dxn = dx / norm
        dyn = dy / norm
        dzn = dz / norm
        f = xar[...] + xbr[...]
        t = _silu(f + rad * w1cr[...] + b1r[...])
        ef = _silu(_dot(t, w2r[...]) + b2r[...])
        g2 = _silu(_dot(ef, cw1r[...]) + cb1r[...])
        cm = jnp.sum(g2 * cw2r[...], axis=1, keepdims=True)
        sel = (lax.broadcasted_iota(jnp.int32, (GPB, BE), 1) // 128
               == lax.broadcasted_iota(jnp.int32, (GPB, BE), 0)).astype(_F32)
        zero = jnp.zeros((), _F32)
        def _seldot(v):
            return jnp.dot(sel, jnp.where(msk, v, zero),
                           preferred_element_type=_F32,
                           precision=lax.Precision.HIGHEST)

        yr[...] = ef
        txr[...] = _seldot(dxn * cm)
        tyr[...] = _seldot(dyn * cm)
        tzr[...] = _seldot(dzn * cm)

    return pl.pallas_call(
        body,
        grid=(EP // BE,),
        in_specs=[pl.BlockSpec((BE, HID), lambda i: (i, 0)),
                  pl.BlockSpec((BE, HID), lambda i: (i, 0)),
                  pl.BlockSpec((GPB, 128), lambda i: (i, 0)),
                  pl.BlockSpec((GPB, 128), lambda i: (i, 0)),
                  pl.BlockSpec((GPB, 128), lambda i: (i, 0)),
                  _full((1, HID)), _full((1, HID)),
                  _full((HID, HID)), _full((1, HID)),
                  _full((HID, HID)), _full((1, HID)),
                  _full((1, HID))],
        out_specs=[pl.BlockSpec((BE, HID), lambda i: (i, 0)),
                   pl.BlockSpec((GPB, 128), lambda i: (i, 0)),
                   pl.BlockSpec((GPB, 128), lambda i: (i, 0)),
                   pl.BlockSpec((GPB, 128), lambda i: (i, 0))],
        out_shape=[jax.ShapeDtypeStruct((EP, HID), _F32),
                   jax.ShapeDtypeStruct((GRP, 128), _F32),
                   jax.ShapeDtypeStruct((GRP, 128), _F32),
                   jax.ShapeDtypeStruct((GRP, 128), _F32)],
    )(xa, xb, dxp, dyp, dzp, w1c, b1, w2T, b2, cw1T, cb1, cw2)


def _tc_node(h, c4, pf, pc4, nw1aT, nw1bT, nb1, nw2T, nb2, wnaT, wnbT):
    def body(hr, cr, pfr, pcr, w1a, w1b, b1r, w2r, b2r, war, wbr,
             hn_ref, cn_ref, a_ref, b_ref):
        aggh = pfr[0] + pfr[1]
        aggc = pcr[0] + pcr[1]
        hcur = hr[...]
        u = _silu(_dot(hcur, w1a[...]) + _dot(aggh, w1b[...]) + b1r[...])
        hn = hcur + _dot(u, w2r[...]) + b2r[...]
        hn_ref[...] = hn
        cn_ref[...] = cr[...] + aggc
        a_ref[...] = _dot(hn, war[...])
        b_ref[...] = _dot(hn, wbr[...])

    return pl.pallas_call(
        body,
        grid=(NT // BN,),
        in_specs=[pl.BlockSpec((BN, HID), lambda i: (i, 0)),
                  pl.BlockSpec((BN, 4), lambda i: (i, 0)),
                  pl.BlockSpec((NC, BN, HID), lambda i: (0, i, 0)),
                  pl.BlockSpec((NC, BN, 4), lambda i: (0, i, 0)),
                  _full((HID, HID)), _full((HID, HID)), _full((1, HID)),
                  _full((HID, HID)), _full((1, HID)),
                  _full((HID, HID)), _full((HID, HID))],
        out_specs=[pl.BlockSpec((BN, HID), lambda i: (i, 0)),
                   pl.BlockSpec((BN, 4), lambda i: (i, 0)),
                   pl.BlockSpec((BN, HID), lambda i: (i, 0)),
                   pl.BlockSpec((BN, HID), lambda i: (i, 0))],
        out_shape=[jax.ShapeDtypeStruct((NT, HID), _F32),
                   jax.ShapeDtypeStruct((NT, 4), _F32),
                   jax.ShapeDtypeStruct((NT, HID), _F32),
                   jax.ShapeDtypeStruct((NT, HID), _F32)],
    )(h, c4, pf, pc4, nw1aT, nw1bT, nb1, nw2T, nb2, wnaT, wnbT)


def _tc_node_last(h, c4, pf, pc4, nw1aT, nw1bT, nb1, nw2T, nb2, eoutT, eoutb):
    def body(hr, cr, pfr, pcr, w1a, w1b, b1r, w2r, b2r, wor, bor,
             ho_ref, cn_ref):
        aggh = pfr[0] + pfr[1]
        aggc = pcr[0] + pcr[1]
        hcur = hr[...]
        u = _silu(_dot(hcur, w1a[...]) + _dot(aggh, w1b[...]) + b1r[...])
        hn = hcur + _dot(u, w2r[...]) + b2r[...]
        ho_ref[...] = _dot(hn, wor[...]) + bor[...]
        cn_ref[...] = cr[...] + aggc

    return pl.pallas_call(
        body,
        grid=(NT // BN,),
        in_specs=[pl.BlockSpec((BN, HID), lambda i: (i, 0)),
                  pl.BlockSpec((BN, 4), lambda i: (i, 0)),
                  pl.BlockSpec((NC, BN, HID), lambda i: (0, i, 0)),
                  pl.BlockSpec((NC, BN, 4), lambda i: (0, i, 0)),
                  _full((HID, HID)), _full((HID, HID)), _full((1, HID)),
                  _full((HID, HID)), _full((1, HID)),
                  _full((HID, D)), _full((1, D))],
        out_specs=[pl.BlockSpec((BN, D), lambda i: (i, 0)),
                   pl.BlockSpec((BN, 4), lambda i: (i, 0))],
        out_shape=[jax.ShapeDtypeStruct((NT, D), _F32),
                   jax.ShapeDtypeStruct((NT, 4), _F32)],
    )(h, c4, pf, pc4, nw1aT, nw1bT, nb1, nw2T, nb2, eoutT, eoutb)


def kernel(h, edge_index, coord, params):
    row = edge_index[0]
    col = edge_index[1]
    npad = EP - E
    padi = N + (jnp.arange(npad, dtype=jnp.int32) % (NT - N))
    rowg = jnp.concatenate([row, padi]).reshape(GRP, 128)
    colg = jnp.concatenate([col, padi]).reshape(GRP, 128)

    h0p = jnp.zeros((NT, D), _F32).at[:N].set(h)
    c4 = jnp.zeros((NT, 4), _F32).at[:N, :3].set(coord)
    zrows = jnp.zeros((128, HID), _F32)
    zflat = jnp.zeros((ZPT,), _F32)

    einT = params['emb_in_W'].T
    einb = params['emb_in_b'].reshape(1, HID)
    eoutT = params['emb_out_W'].T
    eoutb = params['emb_out_b'].reshape(1, D)
    lps = params['layers']

    def w_edge_split(lp):
        w1 = lp['edge_W1']
        return (w1[:, :HID].T, w1[:, HID:2 * HID].T,
                w1[:, 2 * HID].reshape(1, HID))

    w1aT0, w1bT0, _ = w_edge_split(lps[0])
    hcur, a, b = _tc_pre(h0p, einT, einb, w1aT0, w1bT0)

    for li in range(NL):
        lp = lps[li]
        _, _, w1c = w_edge_split(lp)
        cflat = c4.reshape(-1)
        xa, xb, dxp, dyp, dzp = _sc_gather(a, b, cflat, rowg, colg)
        yf, txp, typ, tzp = _tc_edge(xa, xb, dxp, dyp, dzp, w1c,
                                     lp['edge_b1'].reshape(1, HID),
                                     lp['edge_W2'].T,
                                     lp['edge_b2'].reshape(1, HID),
                                     lp['coord_W1'].T,
                                     lp['coord_b1'].reshape(1, HID),
                                     lp['coord_W2'].reshape(1, HID))
        pf, pc = _sc_scatter(yf, txp.reshape(GRP8, 8, 128),
                             typ.reshape(GRP8, 8, 128),
                             tzp.reshape(GRP8, 8, 128), rowg, zrows, zflat)
        pc4 = pc.reshape(NC, NT, 4)
        nw1aT = lp['node_W1'][:, :HID].T
        nw1bT = lp['node_W1'][:, HID:].T
        nb1 = lp['node_b1'].reshape(1, HID)
        nw2T = lp['node_W2'].T
        nb2 = lp['node_b2'].reshape(1, HID)
        if li + 1 < NL:
            wnaT, wnbT, _ = w_edge_split(lps[li + 1])
            hcur, c4, a, b = _tc_node(hcur, c4, pf, pc4, nw1aT, nw1bT,
                                      nb1, nw2T, nb2, wnaT, wnbT)
        else:
            hout, c4 = _tc_node_last(hcur, c4, pf, pc4, nw1aT, nw1bT,
                                     nb1, nw2T, nb2, eoutT, eoutb)

    return (hout[:N], c4[:N, :3])


# speedup vs baseline: 1.1720x; 1.1720x over previous
import functools

import jax
import jax.numpy as jnp
from jax import lax
from jax.experimental import pallas as pl
from jax.experimental.pallas import tpu as pltpu
from jax.experimental.pallas import tpu_sc as plsc

N = 10000
E = 320000
D = 128
HID = 128
NL = 4
EPS = 1e-8

NT = 10240
EP = 327680
GRP = EP // 128
NC = 2
NS = 16
NW = NC * NS
GPW = GRP // NW
GPC = GRP // NC
RPT = NT // NS
BN = 512
BE = 1024
GPB = BE // 128
WID2 = HID + 16

_F32 = jnp.float32


def _silu(x):
    return x * jax.nn.sigmoid(x)


def _dot(a, b):
    return jnp.dot(a, b, preferred_element_type=_F32)


def _sc_gather(a, b, cflat, rowg, colg):
    mesh = plsc.VectorSubcoreMesh(core_axis_name="c", subcore_axis_name="s")

    @functools.partial(
        pl.kernel,
        out_type=[jax.ShapeDtypeStruct((EP, HID), _F32),
                  jax.ShapeDtypeStruct((EP, HID), _F32),
                  jax.ShapeDtypeStruct((GRP, 128), _F32),
                  jax.ShapeDtypeStruct((GRP, 128), _F32),
                  jax.ShapeDtypeStruct((GRP, 128), _F32)],
        mesh=mesh,
        scratch_types=[pltpu.VMEM((GPW, 128), jnp.int32),
                       pltpu.VMEM((GPW, 128), jnp.int32),
                       [pltpu.VMEM((128, HID), _F32) for _ in range(2)],
                       [pltpu.VMEM((128, HID), _F32) for _ in range(2)],
                       [[pltpu.VMEM((128,), jnp.int32) for _ in range(6)]
                        for _ in range(2)],
                       [[pltpu.VMEM((128,), _F32) for _ in range(6)]
                        for _ in range(2)],
                       [pltpu.VMEM((GPW, 128), _F32) for _ in range(3)],
                       [pltpu.SemaphoreType.DMA for _ in range(2)],
                       [pltpu.SemaphoreType.DMA for _ in range(2)],
                       [pltpu.SemaphoreType.DMA for _ in range(2)],
                       [pltpu.SemaphoreType.DMA for _ in range(2)],
                       [pltpu.SemaphoreType.DMA for _ in range(2)]],
    )
    def k(ar, br, cfr, rowr, colr, xa, xb, dxh, dyh, dzh,
          idxr, idxc, bufa, bufb, ixs, cbs, dacc,
          sema, semb, semc, semsa, semsb):
        wid = lax.axis_index("s") * NC + lax.axis_index("c")
        g0 = wid * GPW
        pltpu.sync_copy(rowr.at[pl.ds(g0, GPW)], idxr)
        pltpu.sync_copy(colr.at[pl.ds(g0, GPW)], idxc)
        dxa, dya, dza = dacc

        def fire(g, p):
            ixr0, ixr1, ixr2, ixc0, ixc1, ixc2 = ixs[p]
            for j in range(8):
                sl = pl.ds(j * 16, 16)
                r4 = idxr[g, sl] * 4
                c4 = idxc[g, sl] * 4
                ixr0[sl] = r4
                ixr1[sl] = r4 + 1
                ixr2[sl] = r4 + 2
                ixc0[sl] = c4
                ixc1[sl] = c4 + 1
                ixc2[sl] = c4 + 2
            pltpu.async_copy(ar.at[idxr.at[g]], bufa[p], sema[p])
            pltpu.async_copy(br.at[idxc.at[g]], bufb[p], semb[p])
            brx, bry, brz, bcx, bcy, bcz = cbs[p]
            pltpu.async_copy(cfr.at[ixr0], brx, semc[p])
            pltpu.async_copy(cfr.at[ixr1], bry, semc[p])
            pltpu.async_copy(cfr.at[ixr2], brz, semc[p])
            pltpu.async_copy(cfr.at[ixc0], bcx, semc[p])
            pltpu.async_copy(cfr.at[ixc1], bcy, semc[p])
            pltpu.async_copy(cfr.at[ixc2], bcz, semc[p])

        fire(0, 0)

        def body(s, carry):
            for p in range(2):
                g = s * 2 + p
                q = 1 - p

                @pl.when(g >= 1)
                def _():
                    pltpu.make_async_copy(bufa[q], xa.at[pl.ds(0, 128)],
                                          semsa[q]).wait()
                    pltpu.make_async_copy(bufb[q], xb.at[pl.ds(0, 128)],
                                          semsb[q]).wait()

                @pl.when(g + 1 < GPW)
                def _():
                    fire(g + 1, q)

                pltpu.make_async_copy(ar.at[idxr.at[g]], bufa[p],
                                      sema[p]).wait()
                pltpu.make_async_copy(br.at[idxc.at[g]], bufb[p],
                                      semb[p]).wait()
                brx, bry, brz, bcx, bcy, bcz = cbs[p]
                for cb2 in cbs[p]:
                    pltpu.make_async_copy(cfr.at[pl.ds(0, 128)], cb2,
                                          semc[p]).wait()
                for j in range(8):
                    sl = pl.ds(j * 16, 16)
                    dxa[g, sl] = brx[sl] - bcx[sl]
                    dya[g, sl] = bry[sl] - bcy[sl]
                    dza[g, sl] = brz[sl] - bcz[sl]
                base = (g0 + g) * 128
                pltpu.async_copy(bufa[p], xa.at[pl.ds(base, 128)], semsa[p])
                pltpu.async_copy(bufb[p], xb.at[pl.ds(base, 128)], semsb[p])
            return carry

        lax.fori_loop(0, GPW // 2, body, 0)
        pltpu.make_async_copy(bufa[1], xa.at[pl.ds(0, 128)], semsa[1]).wait()
        pltpu.make_async_copy(bufb[1], xb.at[pl.ds(0, 128)], semsb[1]).wait()
        pltpu.sync_copy(dxa, dxh.at[pl.ds(g0, GPW)])
        pltpu.sync_copy(dya, dyh.at[pl.ds(g0, GPW)])
        pltpu.sync_copy(dza, dzh.at[pl.ds(g0, GPW)])

    return k(a, b, cflat, rowg, colg)


NT4 = NT * 4
ZPT = NT4 // NS
GRP8 = GRP // 8
SPW = GPW // 8


def _sc_scatter(yf, txp, typ, tzp, rowg, zrows, zflat):
    mesh = plsc.VectorSubcoreMesh(core_axis_name="c", subcore_axis_name="s")

    @functools.partial(
        pl.kernel,
        out_type=[jax.ShapeDtypeStruct((NC, NT, HID), _F32),
                  jax.ShapeDtypeStruct((NC, NT4), _F32)],
        mesh=mesh,
        scratch_types=[pltpu.VMEM((8, 128), jnp.int32),
                       [pltpu.VMEM((128, HID), _F32) for _ in range(2)],
                       [pltpu.VMEM((8, 128), _F32) for _ in range(3)],
                       [pltpu.VMEM((128,), jnp.int32) for _ in range(3)],
                       pltpu.VMEM_SHARED((NT, HID), _F32),
                       pltpu.VMEM_SHARED((NT4,), _F32),
                       [pltpu.SemaphoreType.DMA for _ in range(2)],
                       [pltpu.SemaphoreType.DMA for _ in range(2)]],
    )
    def k(yr, txr, tyr, tzr, rowr, zr, zfr, outf, outc,
          idx8, buf, tvs, ixs, accf, accc, semr, semw):
        cid = lax.axis_index("c")
        sid = lax.axis_index("s")
        txv, tyv, tzv = tvs
        ix0, ix1, ix2 = ixs
        for j in range(RPT // 128):
            pltpu.sync_copy(zr, accf.at[pl.ds(sid * RPT + j * 128, 128)])
        pltpu.sync_copy(zfr, accc.at[pl.ds(sid * ZPT, ZPT)])
        plsc.subcore_barrier()

        g0 = cid * GPC + sid * GPW
        s0 = g0 // 8
        pltpu.async_copy(yr.at[pl.ds(g0 * 128, 128)], buf[0], semr[0])

        def body(s, carry):
            @pl.when(s >= 1)
            def _():
                pltpu.make_async_copy(buf[1], accf.at[pl.ds(0, 128)],
                                      semw[1]).wait()

            pltpu.sync_copy(rowr.at[pl.ds(g0 + s * 8, 8)], idx8)
            pltpu.sync_copy(txr.at[s0 + s], txv)
            pltpu.sync_copy(tyr.at[s0 + s], tyv)
            pltpu.sync_copy(tzr.at[s0 + s], tzv)
            for j8 in range(8):
                g = s * 8 + j8
                p = j8 % 2
                q = 1 - p

                if j8 >= 1:
                    pltpu.make_async_copy(buf[q], accf.at[pl.ds(0, 128)],
                                          semw[q]).wait()

                @pl.when(g + 1 < GPW)
                def _():
                    pltpu.async_copy(yr.at[pl.ds((g0 + g + 1) * 128, 128)],
                                     buf[q], semr[q])

                pltpu.make_async_copy(yr.at[pl.ds(0, 128)], buf[p],
                                      semr[p]).wait()
                pltpu.async_copy(buf[p], accf.at[idx8.at[j8]], semw[p],
                                add=True)
                for j in range(8):
                    sl = pl.ds(j * 16, 16)
                    r4 = idx8[j8, sl] * 4
                    ix0[sl] = r4
                    ix1[sl] = r4 + 1
                    ix2[sl] = r4 + 2
                pltpu.sync_copy(txv.at[j8], accc.at[ix0], add=True)
                pltpu.sync_copy(tyv.at[j8], accc.at[ix1], add=True)
                pltpu.sync_copy(tzv.at[j8], accc.at[ix2], add=True)
            return carry

        lax.fori_loop(0, SPW, body, 0)
        pltpu.make_async_copy(buf[1], accf.at[pl.ds(0, 128)], semw[1]).wait()
        plsc.subcore_barrier()
        for j in range(RPT // 128):
            r0 = sid * RPT + j * 128
            pltpu.sync_copy(accf.at[pl.ds(r0, 128)],
                            outf.at[cid, pl.ds(r0, 128)])
        pltpu.sync_copy(accc.at[pl.ds(sid * ZPT, ZPT)],
                        outc.at[cid, pl.ds(sid * ZPT, ZPT)])

    return k(yf, txp, typ, tzp, rowg, zrows, zflat)


def _full(shape):
    return pl.BlockSpec(shape, lambda i: (0,) * len(shape))


def _col_from_planar(p, ei_mask):
    rep = jnp.concatenate(
        [jnp.broadcast_to(p[s:s + 1, :], (128, 128)) for s in range(GPB)],
        axis=0)
    return jnp.sum(jnp.where(ei_mask, rep, 0.0), axis=1, keepdims=True)


def _tc_pre(h0p, einT, einb, w1aT, w1bT):
    def body(h_ref, wt, bt, wa, wb, h1_ref, a_ref, b_ref):
        h1 = _dot(h_ref[...], wt[...]) + bt[...]
        h1_ref[...] = h1
        a_ref[...] = _dot(h1, wa[...])
        b_ref[...] = _dot(h1, wb[...])

    return pl.pallas_call(
        body,
        grid=(NT // BN,),
        in_specs=[pl.BlockSpec((BN, D), lambda i: (i, 0)),
                  _full((D, HID)), _full((1, HID)),
                  _full((HID, HID)), _full((HID, HID))],
        out_specs=[pl.BlockSpec((BN, HID), lambda i: (i, 0)),
                   pl.BlockSpec((BN, HID), lambda i: (i, 0)),
                   pl.BlockSpec((BN, HID), lambda i: (i, 0))],
        out_shape=[jax.ShapeDtypeStruct((NT, HID), _F32),
                   jax.ShapeDtypeStruct((NT, HID), _F32),
                   jax.ShapeDtypeStruct((NT, HID), _F32)],
    )(h0p, einT, einb, w1aT, w1bT)


def _tc_edge(xa, xb, dxp, dyp, dzp, w1c, b1, w2T, b2, cw1T, cb1, cw2):
    def body(xar, xbr, dxr, dyr, dzr, w1cr, b1r, w2r, b2r, cw1r, cb1r,
             cw2r, yr, txr, tyr, tzr):
        li = lax.broadcasted_iota(jnp.int32, (BE, 128), 1)
        ei = lax.broadcasted_iota(jnp.int32, (BE, 128), 0)
        msk = li == (ei % 128)
        dx = _col_from_planar(dxr[...], msk)
        dy = _col_from_planar(dyr[...], msk)
        dz = _col_from_planar(dzr[...], msk)
        rad = dx * dx + dy * dy + dz * dz
        norm = jnp.sqrt(rad) + EPS
        dxn = dx / norm
        dyn = dy / norm
        dzn = dz / norm
        f = xar[...] + xbr[...]
        t = _silu(f + rad * w1cr[...] + b1r[...])
        ef = _silu(_dot(t, w2r[...]) + b2r[...])
        g2 = _silu(_dot(ef, cw1r[...]) + cb1r[...])
        cm = jnp.sum(g2 * cw2r[...], axis=1, keepdims=True)
        sel = (lax.broadcasted_iota(jnp.int32, (GPB, BE), 1) // 128
               == lax.broadcasted_iota(jnp.int32, (GPB, BE), 0)).astype(_F32)
        zero = jnp.zeros((), _F32)
        def _seldot(v):
            return jnp.dot(sel, jnp.where(msk, v, zero),
                           preferred_element_type=_F32)

        yr[...] = ef
        txr[...] = _seldot(dxn * cm)
        tyr[...] = _seldot(dyn * cm)
        tzr[...] = _seldot(dzn * cm)

    return pl.pallas_call(
        body,
        grid=(EP // BE,),
        in_specs=[pl.BlockSpec((BE, HID), lambda i: (i, 0)),
                  pl.BlockSpec((BE, HID), lambda i: (i, 0)),
                  pl.BlockSpec((GPB, 128), lambda i: (i, 0)),
                  pl.BlockSpec((GPB, 128), lambda i: (i, 0)),
                  pl.BlockSpec((GPB, 128), lambda i: (i, 0)),
                  _full((1, HID)), _full((1, HID)),
                  _full((HID, HID)), _full((1, HID)),
                  _full((HID, HID)), _full((1, HID)),
                  _full((1, HID))],
        out_specs=[pl.BlockSpec((BE, HID), lambda i: (i, 0)),
                   pl.BlockSpec((GPB, 128), lambda i: (i, 0)),
                   pl.BlockSpec((GPB, 128), lambda i: (i, 0)),
                   pl.BlockSpec((GPB, 128), lambda i: (i, 0))],
        out_shape=[jax.ShapeDtypeStruct((EP, HID), _F32),
                   jax.ShapeDtypeStruct((GRP, 128), _F32),
                   jax.ShapeDtypeStruct((GRP, 128), _F32),
                   jax.ShapeDtypeStruct((GRP, 128), _F32)],
    )(xa, xb, dxp, dyp, dzp, w1c, b1, w2T, b2, cw1T, cb1, cw2)


def _tc_node(h, c4, pf, pc4, nw1aT, nw1bT, nb1, nw2T, nb2, wnaT, wnbT):
    def body(hr, cr, pfr, pcr, w1a, w1b, b1r, w2r, b2r, war, wbr,
             hn_ref, cn_ref, a_ref, b_ref):
        aggh = pfr[0] + pfr[1]
        aggc = pcr[0] + pcr[1]
        hcur = hr[...]
        u = _silu(_dot(hcur, w1a[...]) + _dot(aggh, w1b[...]) + b1r[...])
        hn = hcur + _dot(u, w2r[...]) + b2r[...]
        hn_ref[...] = hn
        cn_ref[...] = cr[...] + aggc
        a_ref[...] = _dot(hn, war[...])
        b_ref[...] = _dot(hn, wbr[...])

    return pl.pallas_call(
        body,
        grid=(NT // BN,),
        in_specs=[pl.BlockSpec((BN, HID), lambda i: (i, 0)),
                  pl.BlockSpec((BN, 4), lambda i: (i, 0)),
                  pl.BlockSpec((NC, BN, HID), lambda i: (0, i, 0)),
                  pl.BlockSpec((NC, BN, 4), lambda i: (0, i, 0)),
                  _full((HID, HID)), _full((HID, HID)), _full((1, HID)),
                  _full((HID, HID)), _full((1, HID)),
                  _full((HID, HID)), _full((HID, HID))],
        out_specs=[pl.BlockSpec((BN, HID), lambda i: (i, 0)),
                   pl.BlockSpec((BN, 4), lambda i: (i, 0)),
                   pl.BlockSpec((BN, HID), lambda i: (i, 0)),
                   pl.BlockSpec((BN, HID), lambda i: (i, 0))],
        out_shape=[jax.ShapeDtypeStruct((NT, HID), _F32),
                   jax.ShapeDtypeStruct((NT, 4), _F32),
                   jax.ShapeDtypeStruct((NT, HID), _F32),
                   jax.ShapeDtypeStruct((NT, HID), _F32)],
    )(h, c4, pf, pc4, nw1aT, nw1bT, nb1, nw2T, nb2, wnaT, wnbT)


def _tc_node_last(h, c4, pf, pc4, nw1aT, nw1bT, nb1, nw2T, nb2, eoutT, eoutb):
    def body(hr, cr, pfr, pcr, w1a, w1b, b1r, w2r, b2r, wor, bor,
             ho_ref, cn_ref):
        aggh = pfr[0] + pfr[1]
        aggc = pcr[0] + pcr[1]
        hcur = hr[...]
        u = _silu(_dot(hcur, w1a[...]) + _dot(aggh, w1b[...]) + b1r[...])
        hn = hcur + _dot(u, w2r[...]) + b2r[...]
        ho_ref[...] = _dot(hn, wor[...]) + bor[...]
        cn_ref[...] = cr[...] + aggc

    return pl.pallas_call(
        body,
        grid=(NT // BN,),
        in_specs=[pl.BlockSpec((BN, HID), lambda i: (i, 0)),
                  pl.BlockSpec((BN, 4), lambda i: (i, 0)),
                  pl.BlockSpec((NC, BN, HID), lambda i: (0, i, 0)),
                  pl.BlockSpec((NC, BN, 4), lambda i: (0, i, 0)),
                  _full((HID, HID)), _full((HID, HID)), _full((1, HID)),
                  _full((HID, HID)), _full((1, HID)),
                  _full((HID, D)), _full((1, D))],
        out_specs=[pl.BlockSpec((BN, D), lambda i: (i, 0)),
                   pl.BlockSpec((BN, 4), lambda i: (i, 0))],
        out_shape=[jax.ShapeDtypeStruct((NT, D), _F32),
                   jax.ShapeDtypeStruct((NT, 4), _F32)],
    )(h, c4, pf, pc4, nw1aT, nw1bT, nb1, nw2T, nb2, eoutT, eoutb)


def kernel(h, edge_index, coord, params):
    row = edge_index[0]
    col = edge_index[1]
    npad = EP - E
    padi = N + (jnp.arange(npad, dtype=jnp.int32) % (NT - N))
    rowg = jnp.concatenate([row, padi]).reshape(GRP, 128)
    colg = jnp.concatenate([col, padi]).reshape(GRP, 128)

    h0p = jnp.zeros((NT, D), _F32).at[:N].set(h)
    c4 = jnp.zeros((NT, 4), _F32).at[:N, :3].set(coord)
    zrows = jnp.zeros((128, HID), _F32)
    zflat = jnp.zeros((ZPT,), _F32)

    einT = params['emb_in_W'].T
    einb = params['emb_in_b'].reshape(1, HID)
    eoutT = params['emb_out_W'].T
    eoutb = params['emb_out_b'].reshape(1, D)
    lps = params['layers']

    def w_edge_split(lp):
        w1 = lp['edge_W1']
        return (w1[:, :HID].T, w1[:, HID:2 * HID].T,
                w1[:, 2 * HID].reshape(1, HID))

    w1aT0, w1bT0, _ = w_edge_split(lps[0])
    hcur, a, b = _tc_pre(h0p, einT, einb, w1aT0, w1bT0)

    for li in range(NL):
        lp = lps[li]
        _, _, w1c = w_edge_split(lp)
        cflat = c4.reshape(-1)
        xa, xb, dxp, dyp, dzp = _sc_gather(a, b, cflat, rowg, colg)
        yf, txp, typ, tzp = _tc_edge(xa, xb, dxp, dyp, dzp, w1c,
                                     lp['edge_b1'].reshape(1, HID),
                                     lp['edge_W2'].T,
                                     lp['edge_b2'].reshape(1, HID),
                                     lp['coord_W1'].T,
                                     lp['coord_b1'].reshape(1, HID),
                                     lp['coord_W2'].reshape(1, HID))
        pf, pc = _sc_scatter(yf, txp.reshape(GRP8, 8, 128),
                             typ.reshape(GRP8, 8, 128),
                             tzp.reshape(GRP8, 8, 128), rowg, zrows, zflat)
        pc4 = pc.reshape(NC, NT, 4)
        nw1aT = lp['node_W1'][:, :HID].T
        nw1bT = lp['node_W1'][:, HID:].T
        nb1 = lp['node_b1'].reshape(1, HID)
        nw2T = lp['node_W2'].T
        nb2 = lp['node_b2'].reshape(1, HID)
        if li + 1 < NL:
            wnaT, wnbT, _ = w_edge_split(lps[li + 1])
            hcur, c4, a, b = _tc_node(hcur, c4, pf, pc4, nw1aT, nw1bT,
                                      nb1, nw2T, nb2, wnaT, wnbT)
        else:
            hout, c4 = _tc_node_last(hcur, c4, pf, pc4, nw1aT, nw1bT,
                                     nb1, nw2T, nb2, eoutT, eoutb)

    return (hout[:N], c4[:N, :3])


# speedup vs baseline: 1.2763x; 1.0890x over previous
import functools

import jax
import jax.numpy as jnp
from jax import lax
from jax.experimental import pallas as pl
from jax.experimental.pallas import tpu as pltpu
from jax.experimental.pallas import tpu_sc as plsc

N = 10000
E = 320000
D = 128
HID = 128
NL = 4
EPS = 1e-8

NT = 10240
EP = 327680
GRP = EP // 128
NC = 2
NS = 16
NW = NC * NS
EPH = EP // 2
GRPH = GRP // 2
GPWG = GRPH // NW
GPW = GRPH // NS
RPT = NT // NS
BN = 512
BE = 1024
GPB = BE // 128

_F32 = jnp.float32


def _silu(x):
    return x * jax.nn.sigmoid(x)


def _dot(a, b):
    return jnp.dot(a, b, preferred_element_type=_F32)


def _sc_gather(a, b, cflat, rowg, colg):
    mesh = plsc.VectorSubcoreMesh(core_axis_name="c", subcore_axis_name="s")

    @functools.partial(
        pl.kernel,
        out_type=[jax.ShapeDtypeStruct((EPH, HID), _F32),
                  jax.ShapeDtypeStruct((EPH, HID), _F32),
                  jax.ShapeDtypeStruct((GRPH, 128), _F32),
                  jax.ShapeDtypeStruct((GRPH, 128), _F32),
                  jax.ShapeDtypeStruct((GRPH, 128), _F32)],
        mesh=mesh,
        scratch_types=[pltpu.VMEM((GPWG, 128), jnp.int32),
                       pltpu.VMEM((GPWG, 128), jnp.int32),
                       [pltpu.VMEM((128, HID), _F32) for _ in range(2)],
                       [pltpu.VMEM((128, HID), _F32) for _ in range(2)],
                       [[pltpu.VMEM((128,), jnp.int32) for _ in range(6)]
                        for _ in range(2)],
                       [[pltpu.VMEM((128,), _F32) for _ in range(6)]
                        for _ in range(2)],
                       [pltpu.VMEM((GPWG, 128), _F32) for _ in range(3)],
                       [pltpu.SemaphoreType.DMA for _ in range(2)],
                       [pltpu.SemaphoreType.DMA for _ in range(2)],
                       [pltpu.SemaphoreType.DMA for _ in range(2)],
                       [pltpu.SemaphoreType.DMA for _ in range(2)],
                       [pltpu.SemaphoreType.DMA for _ in range(2)]],
    )
    def k(ar, br, cfr, rowr, colr, xa, xb, dxh, dyh, dzh,
          idxr, idxc, bufa, bufb, ixs, cbs, dacc,
          sema, semb, semc, semsa, semsb):
        wid = lax.axis_index("s") * NC + lax.axis_index("c")
        g0 = wid * GPWG
        pltpu.sync_copy(rowr.at[pl.ds(g0, GPWG)], idxr)
        pltpu.sync_copy(colr.at[pl.ds(g0, GPWG)], idxc)
        dxa, dya, dza = dacc

        def fire(g, p):
            ixr0, ixr1, ixr2, ixc0, ixc1, ixc2 = ixs[p]
            for j in range(8):
                sl = pl.ds(j * 16, 16)
                r4 = idxr[g, sl] * 4
                c4 = idxc[g, sl] * 4
                ixr0[sl] = r4
                ixr1[sl] = r4 + 1
                ixr2[sl] = r4 + 2
                ixc0[sl] = c4
                ixc1[sl] = c4 + 1
                ixc2[sl] = c4 + 2
            pltpu.async_copy(ar.at[idxr.at[g]], bufa[p], sema[p])
            pltpu.async_copy(br.at[idxc.at[g]], bufb[p], semb[p])
            brx, bry, brz, bcx, bcy, bcz = cbs[p]
            pltpu.async_copy(cfr.at[ixr0], brx, semc[p])
            pltpu.async_copy(cfr.at[ixr1], bry, semc[p])
            pltpu.async_copy(cfr.at[ixr2], brz, semc[p])
            pltpu.async_copy(cfr.at[ixc0], bcx, semc[p])
            pltpu.async_copy(cfr.at[ixc1], bcy, semc[p])
            pltpu.async_copy(cfr.at[ixc2], bcz, semc[p])

        fire(0, 0)

        def body(s, carry):
            for p in range(2):
                g = s * 2 + p
                q = 1 - p

                @pl.when(g >= 1)
                def _():
                    pltpu.make_async_copy(bufa[q], xa.at[pl.ds(0, 128)],
                                          semsa[q]).wait()
                    pltpu.make_async_copy(bufb[q], xb.at[pl.ds(0, 128)],
                                          semsb[q]).wait()

                @pl.when(g + 1 < GPWG)
                def _():
                    fire(g + 1, q)

                pltpu.make_async_copy(ar.at[idxr.at[g]], bufa[p],
                                      sema[p]).wait()
                pltpu.make_async_copy(br.at[idxc.at[g]], bufb[p],
                                      semb[p]).wait()
                brx, bry, brz, bcx, bcy, bcz = cbs[p]
                for cb2 in cbs[p]:
                    pltpu.make_async_copy(cfr.at[pl.ds(0, 128)], cb2,
                                          semc[p]).wait()
                for j in range(8):
                    sl = pl.ds(j * 16, 16)
                    dxa[g, sl] = brx[sl] - bcx[sl]
                    dya[g, sl] = bry[sl] - bcy[sl]
                    dza[g, sl] = brz[sl] - bcz[sl]
                base = (g0 + g) * 128
                pltpu.async_copy(bufa[p], xa.at[pl.ds(base, 128)], semsa[p])
                pltpu.async_copy(bufb[p], xb.at[pl.ds(base, 128)], semsb[p])
            return carry

        lax.fori_loop(0, GPWG // 2, body, 0)
        pltpu.make_async_copy(bufa[1], xa.at[pl.ds(0, 128)], semsa[1]).wait()
        pltpu.make_async_copy(bufb[1], xb.at[pl.ds(0, 128)], semsb[1]).wait()
        pltpu.sync_copy(dxa, dxh.at[pl.ds(g0, GPWG)])
        pltpu.sync_copy(dya, dyh.at[pl.ds(g0, GPWG)])
        pltpu.sync_copy(dza, dzh.at[pl.ds(g0, GPWG)])

    return k(a, b, cflat, rowg, colg)


NT4 = NT * 4
ZPT = NT4 // NS
GRP8H = GRPH // 8
SPW = GPW // 8


def _sc_scatter(yf1, yf2, t1, t2, rowg1, rowg2, zrows, zflat):
    mesh = plsc.VectorSubcoreMesh(core_axis_name="c", subcore_axis_name="s")

    @functools.partial(
        pl.kernel,
        out_type=[jax.ShapeDtypeStruct((NC, NT, HID), _F32),
                  jax.ShapeDtypeStruct((NC, NT4), _F32)],
        mesh=mesh,
        scratch_types=[pltpu.VMEM((8, 128), jnp.int32),
                       [pltpu.VMEM((128, HID), _F32) for _ in range(2)],
                       [pltpu.VMEM((8, 128), _F32) for _ in range(3)],
                       [pltpu.VMEM((128,), jnp.int32) for _ in range(3)],
                       pltpu.VMEM_SHARED((NT, HID), _F32),
                       pltpu.VMEM_SHARED((NT4,), _F32),
                       [pltpu.SemaphoreType.DMA for _ in range(2)],
                       [pltpu.SemaphoreType.DMA for _ in range(2)]],
    )
    def k(y1, tx1, ty1, tz1, y2, tx2, ty2, tz2, r1, r2, zr, zfr, outf, outc,
          idx8, buf, tvs, ixs, accf, accc, semr, semw):
        cid = lax.axis_index("c")
        sid = lax.axis_index("s")
        txv, tyv, tzv = tvs
        ix0, ix1, ix2 = ixs
        for j in range(RPT // 128):
            pltpu.sync_copy(zr, accf.at[pl.ds(sid * RPT + j * 128, 128)])
        pltpu.sync_copy(zfr, accc.at[pl.ds(sid * ZPT, ZPT)])
        plsc.subcore_barrier()

        g0 = sid * GPW
        s0 = g0 // 8

        def run(yr, txr, tyr, tzr, rowr):
            pltpu.async_copy(yr.at[pl.ds(g0 * 128, 128)], buf[0], semr[0])

            def body(s, carry):
                @pl.when(s >= 1)
                def _():
                    pltpu.make_async_copy(buf[1], accf.at[pl.ds(0, 128)],
                                          semw[1]).wait()

                pltpu.sync_copy(rowr.at[pl.ds(g0 + s * 8, 8)], idx8)
                pltpu.sync_copy(txr.at[s0 + s], txv)
                pltpu.sync_copy(tyr.at[s0 + s], tyv)
                pltpu.sync_copy(tzr.at[s0 + s], tzv)
                for j8 in range(8):
                    g = s * 8 + j8
                    p = j8 % 2
                    q = 1 - p

                    if j8 >= 1:
                        pltpu.make_async_copy(buf[q], accf.at[pl.ds(0, 128)],
                                              semw[q]).wait()

                    @pl.when(g + 1 < GPW)
                    def _():
                        pltpu.async_copy(
                            yr.at[pl.ds((g0 + g + 1) * 128, 128)],
                            buf[q], semr[q])

                    pltpu.make_async_copy(yr.at[pl.ds(0, 128)], buf[p],
                                          semr[p]).wait()
                    pltpu.async_copy(buf[p], accf.at[idx8.at[j8]], semw[p],
                                    add=True)
                    for j in range(8):
                        sl = pl.ds(j * 16, 16)
                        r4 = idx8[j8, sl] * 4
                        ix0[sl] = r4
                        ix1[sl] = r4 + 1
                        ix2[sl] = r4 + 2
                    pltpu.sync_copy(txv.at[j8], accc.at[ix0], add=True)
                    pltpu.sync_copy(tyv.at[j8], accc.at[ix1], add=True)
                    pltpu.sync_copy(tzv.at[j8], accc.at[ix2], add=True)
                return carry

            lax.fori_loop(0, SPW, body, 0)
            pltpu.make_async_copy(buf[1], accf.at[pl.ds(0, 128)],
                                  semw[1]).wait()

        @pl.when(cid == 0)
        def _():
            run(y1, tx1, ty1, tz1, r1)

        @pl.when(cid == 1)
        def _():
            run(y2, tx2, ty2, tz2, r2)

        plsc.subcore_barrier()
        for j in range(RPT // 128):
            r0 = sid * RPT + j * 128
            pltpu.sync_copy(accf.at[pl.ds(r0, 128)],
                            outf.at[cid, pl.ds(r0, 128)])
        pltpu.sync_copy(accc.at[pl.ds(sid * ZPT, ZPT)],
                        outc.at[cid, pl.ds(sid * ZPT, ZPT)])

    return k(yf1, t1[0], t1[1], t1[2], yf2, t2[0], t2[1], t2[2],
             rowg1, rowg2, zrows, zflat)


def _full(shape):
    return pl.BlockSpec(shape, lambda i: (0,) * len(shape))


def _col_from_planar(p, ei_mask):
    rep = jnp.concatenate(
        [jnp.broadcast_to(p[s:s + 1, :], (128, 128)) for s in range(GPB)],
        axis=0)
    return jnp.sum(jnp.where(ei_mask, rep, 0.0), axis=1, keepdims=True)


def _tc_pre(h0p, einT, einb, w1aT, w1bT):
    def body(h_ref, wt, bt, wa, wb, h1_ref, a_ref, b_ref):
        h1 = _dot(h_ref[...], wt[...]) + bt[...]
        h1_ref[...] = h1
        a_ref[...] = _dot(h1, wa[...])
        b_ref[...] = _dot(h1, wb[...])

    return pl.pallas_call(
        body,
        grid=(NT // BN,),
        in_specs=[pl.BlockSpec((BN, D), lambda i: (i, 0)),
                  _full((D, HID)), _full((1, HID)),
                  _full((HID, HID)), _full((HID, HID))],
        out_specs=[pl.BlockSpec((BN, HID), lambda i: (i, 0)),
                   pl.BlockSpec((BN, HID), lambda i: (i, 0)),
                   pl.BlockSpec((BN, HID), lambda i: (i, 0))],
        out_shape=[jax.ShapeDtypeStruct((NT, HID), _F32),
                   jax.ShapeDtypeStruct((NT, HID), _F32),
                   jax.ShapeDtypeStruct((NT, HID), _F32)],
    )(h0p, einT, einb, w1aT, w1bT)


def _tc_edge(xa, xb, dxp, dyp, dzp, w1c, b1, w2T, b2, cw1T, cb1, cw2):
    def body(xar, xbr, dxr, dyr, dzr, w1cr, b1r, w2r, b2r, cw1r, cb1r,
             cw2r, yr, txr, tyr, tzr):
        li = lax.broadcasted_iota(jnp.int32, (BE, 128), 1)
        ei = lax.broadcasted_iota(jnp.int32, (BE, 128), 0)
        msk = li == (ei % 128)
        dx = _col_from_planar(dxr[...], msk)
        dy = _col_from_planar(dyr[...], msk)
        dz = _col_from_planar(dzr[...], msk)
        rad = dx * dx + dy * dy + dz * dz
        norm = jnp.sqrt(rad) + EPS
        dxn = dx / norm
        dyn = dy / norm
        dzn = dz / norm
        f = xar[...] + xbr[...]
        t = _silu(f + rad * w1cr[...] + b1r[...])
        ef = _silu(_dot(t, w2r[...]) + b2r[...])
        g2 = _silu(_dot(ef, cw1r[...]) + cb1r[...])
        cm = jnp.sum(g2 * cw2r[...], axis=1, keepdims=True)
        sel = (lax.broadcasted_iota(jnp.int32, (GPB, BE), 1) // 128
               == lax.broadcasted_iota(jnp.int32, (GPB, BE), 0)).astype(_F32)
        zero = jnp.zeros((), _F32)
        def _seldot(v):
            return jnp.dot(sel, jnp.where(msk, v, zero),
                           preferred_element_type=_F32)

        yr[...] = ef
        txr[...] = _seldot(dxn * cm)
        tyr[...] = _seldot(dyn * cm)
        tzr[...] = _seldot(dzn * cm)

    return pl.pallas_call(
        body,
        grid=(EPH // BE,),
        in_specs=[pl.BlockSpec((BE, HID), lambda i: (i, 0)),
                  pl.BlockSpec((BE, HID), lambda i: (i, 0)),
                  pl.BlockSpec((GPB, 128), lambda i: (i, 0)),
                  pl.BlockSpec((GPB, 128), lambda i: (i, 0)),
                  pl.BlockSpec((GPB, 128), lambda i: (i, 0)),
                  _full((1, HID)), _full((1, HID)),
                  _full((HID, HID)), _full((1, HID)),
                  _full((HID, HID)), _full((1, HID)),
                  _full((1, HID))],
        out_specs=[pl.BlockSpec((BE, HID), lambda i: (i, 0)),
                   pl.BlockSpec((GPB, 128), lambda i: (i, 0)),
                   pl.BlockSpec((GPB, 128), lambda i: (i, 0)),
                   pl.BlockSpec((GPB, 128), lambda i: (i, 0))],
        out_shape=[jax.ShapeDtypeStruct((EPH, HID), _F32),
                   jax.ShapeDtypeStruct((GRPH, 128), _F32),
                   jax.ShapeDtypeStruct((GRPH, 128), _F32),
                   jax.ShapeDtypeStruct((GRPH, 128), _F32)],
    )(xa, xb, dxp, dyp, dzp, w1c, b1, w2T, b2, cw1T, cb1, cw2)


def _tc_node(h, c4, pf, pc4, nw1aT, nw1bT, nb1, nw2T, nb2, wnaT, wnbT):
    def body(hr, cr, pfr, pcr, w1a, w1b, b1r, w2r, b2r, war, wbr,
             hn_ref, cn_ref, a_ref, b_ref):
        aggh = pfr[0] + pfr[1]
        aggc = pcr[0] + pcr[1]
        hcur = hr[...]
        u = _silu(_dot(hcur, w1a[...]) + _dot(aggh, w1b[...]) + b1r[...])
        hn = hcur + _dot(u, w2r[...]) + b2r[...]
        hn_ref[...] = hn
        cn_ref[...] = cr[...] + aggc
        a_ref[...] = _dot(hn, war[...])
        b_ref[...] = _dot(hn, wbr[...])

    return pl.pallas_call(
        body,
        grid=(NT // BN,),
        in_specs=[pl.BlockSpec((BN, HID), lambda i: (i, 0)),
                  pl.BlockSpec((BN, 4), lambda i: (i, 0)),
                  pl.BlockSpec((NC, BN, HID), lambda i: (0, i, 0)),
                  pl.BlockSpec((NC, BN, 4), lambda i: (0, i, 0)),
                  _full((HID, HID)), _full((HID, HID)), _full((1, HID)),
                  _full((HID, HID)), _full((1, HID)),
                  _full((HID, HID)), _full((HID, HID))],
        out_specs=[pl.BlockSpec((BN, HID), lambda i: (i, 0)),
                   pl.BlockSpec((BN, 4), lambda i: (i, 0)),
                   pl.BlockSpec((BN, HID), lambda i: (i, 0)),
                   pl.BlockSpec((BN, HID), lambda i: (i, 0))],
        out_shape=[jax.ShapeDtypeStruct((NT, HID), _F32),
                   jax.ShapeDtypeStruct((NT, 4), _F32),
                   jax.ShapeDtypeStruct((NT, HID), _F32),
                   jax.ShapeDtypeStruct((NT, HID), _F32)],
    )(h, c4, pf, pc4, nw1aT, nw1bT, nb1, nw2T, nb2, wnaT, wnbT)


def _tc_node_last(h, c4, pf, pc4, nw1aT, nw1bT, nb1, nw2T, nb2, eoutT, eoutb):
    def body(hr, cr, pfr, pcr, w1a, w1b, b1r, w2r, b2r, wor, bor,
             ho_ref, cn_ref):
        aggh = pfr[0] + pfr[1]
        aggc = pcr[0] + pcr[1]
        hcur = hr[...]
        u = _silu(_dot(hcur, w1a[...]) + _dot(aggh, w1b[...]) + b1r[...])
        hn = hcur + _dot(u, w2r[...]) + b2r[...]
        ho_ref[...] = _dot(hn, wor[...]) + bor[...]
        cn_ref[...] = cr[...] + aggc

    return pl.pallas_call(
        body,
        grid=(NT // BN,),
        in_specs=[pl.BlockSpec((BN, HID), lambda i: (i, 0)),
                  pl.BlockSpec((BN, 4), lambda i: (i, 0)),
                  pl.BlockSpec((NC, BN, HID), lambda i: (0, i, 0)),
                  pl.BlockSpec((NC, BN, 4), lambda i: (0, i, 0)),
                  _full((HID, HID)), _full((HID, HID)), _full((1, HID)),
                  _full((HID, HID)), _full((1, HID)),
                  _full((HID, D)), _full((1, D))],
        out_specs=[pl.BlockSpec((BN, D), lambda i: (i, 0)),
                   pl.BlockSpec((BN, 4), lambda i: (i, 0))],
        out_shape=[jax.ShapeDtypeStruct((NT, D), _F32),
                   jax.ShapeDtypeStruct((NT, 4), _F32)],
    )(h, c4, pf, pc4, nw1aT, nw1bT, nb1, nw2T, nb2, eoutT, eoutb)


def kernel(h, edge_index, coord, params):
    row = edge_index[0]
    col = edge_index[1]
    npad = EP - E
    padi = N + (jnp.arange(npad, dtype=jnp.int32) % (NT - N))
    rowg = jnp.concatenate([row, padi]).reshape(GRP, 128)
    colg = jnp.concatenate([col, padi]).reshape(GRP, 128)

    h0p = jnp.zeros((NT, D), _F32).at[:N].set(h)
    c4 = jnp.zeros((NT, 4), _F32).at[:N, :3].set(coord)
    zrows = jnp.zeros((128, HID), _F32)
    zflat = jnp.zeros((ZPT,), _F32)

    einT = params['emb_in_W'].T
    einb = params['emb_in_b'].reshape(1, HID)
    eoutT = params['emb_out_W'].T
    eoutb = params['emb_out_b'].reshape(1, D)
    lps = params['layers']

    def w_edge_split(lp):
        w1 = lp['edge_W1']
        return (w1[:, :HID].T, w1[:, HID:2 * HID].T,
                w1[:, 2 * HID].reshape(1, HID))

    w1aT0, w1bT0, _ = w_edge_split(lps[0])
    hcur, a, b = _tc_pre(h0p, einT, einb, w1aT0, w1bT0)

    for li in range(NL):
        lp = lps[li]
        _, _, w1c = w_edge_split(lp)
        cflat = c4.reshape(-1)
        ew = (w1c, lp['edge_b1'].reshape(1, HID), lp['edge_W2'].T,
              lp['edge_b2'].reshape(1, HID), lp['coord_W1'].T,
              lp['coord_b1'].reshape(1, HID), lp['coord_W2'].reshape(1, HID))
        halves = []
        for hh in range(2):
            xa, xb, dxp, dyp, dzp = _sc_gather(a, b, cflat,
                                               rowg[hh * GRPH:(hh + 1) * GRPH],
                                               colg[hh * GRPH:(hh + 1) * GRPH])
            yf, txp, typ, tzp = _tc_edge(xa, xb, dxp, dyp, dzp, *ew)
            halves.append((yf, (txp.reshape(GRP8H, 8, 128),
                                typ.reshape(GRP8H, 8, 128),
                                tzp.reshape(GRP8H, 8, 128))))
        pf, pc = _sc_scatter(halves[0][0], halves[1][0],
                             halves[0][1], halves[1][1],
                             rowg[:GRPH], rowg[GRPH:], zrows, zflat)
        pc4 = pc.reshape(NC, NT, 4)
        nw1aT = lp['node_W1'][:, :HID].T
        nw1bT = lp['node_W1'][:, HID:].T
        nb1 = lp['node_b1'].reshape(1, HID)
        nw2T = lp['node_W2'].T
        nb2 = lp['node_b2'].reshape(1, HID)
        if li + 1 < NL:
            wnaT, wnbT, _ = w_edge_split(lps[li + 1])
            hcur, c4, a, b = _tc_node(hcur, c4, pf, pc4, nw1aT, nw1bT,
                                      nb1, nw2T, nb2, wnaT, wnbT)
        else:
            hout, c4 = _tc_node_last(hcur, c4, pf, pc4, nw1aT, nw1bT,
                                     nb1, nw2T, nb2, eoutT, eoutb)

    return (hout[:N], c4[:N, :3])
